# baseline (speedup 1.0000x reference)
import functools

import jax
import jax.numpy as jnp
from jax import lax
from jax.experimental import pallas as pl
from jax.experimental.pallas import tpu as pltpu
from jax.experimental.pallas import tpu_sc as plsc

_PREC = lax.Precision.DEFAULT
_TAPS2 = ((0, 0), (0, 1), (1, 0), (1, 1))


def _leaky(x):
    return jnp.where(x >= 0, x, 0.01 * x)


def _halo_tiles(xpad, Th, mx):
    B, Hp, Wp, C = xpad.shape
    NT = (Hp - mx) // Th
    return jnp.stack([xpad[:, i * Th:i * Th + Th + mx] for i in range(NT)],
                     axis=1)


def _unphase2(y, Co):
    B, H, W, _ = y.shape
    y = y.reshape(B, H, W, 2, 2, Co).transpose(0, 1, 3, 2, 4, 5)
    return y.reshape(B, 2 * H, 2 * W, Co)


def _tap_conv(xs, w, b, taps, Wo, relu):
    B, NT, Ht, Wp, C = xs.shape
    T = len(taps)
    mx = max(dy for dy, _ in taps)
    Th = Ht - mx
    N = w.shape[1]
    pertap = C >= 128

    def body(x_ref, w_ref, b_ref, o_ref):
        if pertap:
            acc = None
            for t, (dy, dx) in enumerate(taps):
                xt = x_ref[0, 0, dy:dy + Th, dx:dx + Wo, :]
                d = lax.dot_general(xt.reshape(Th * Wo, C),
                                    w_ref[t * C:(t + 1) * C, :],
                                    (((1,), (0,)), ((), ())),
                                    preferred_element_type=jnp.float32,
                                    precision=_PREC)
                acc = d if acc is None else acc + d
        else:
            parts = [x_ref[0, 0, dy:dy + Th, dx:dx + Wo, :]
                     for dy, dx in taps]
            xt = jnp.concatenate(parts, axis=-1).reshape(Th * Wo, T * C)
            acc = lax.dot_general(xt, w_ref[...], (((1,), (0,)), ((), ())),
                                  preferred_element_type=jnp.float32,
                                  precision=_PREC)
        y = acc + b_ref[0, :][None, :]
        if relu:
            y = _leaky(y)
        o_ref[0] = y.reshape(Th, Wo, N)

    return pl.pallas_call(
        body,
        grid=(B, NT),
        in_specs=[
            pl.BlockSpec((1, 1, Ht, Wp, C), lambda bb, r: (bb, r, 0, 0, 0)),
            pl.BlockSpec((T * C, N), lambda bb, r: (0, 0)),
            pl.BlockSpec((1, N), lambda bb, r: (0, 0)),
        ],
        out_specs=pl.BlockSpec((1, Th, Wo, N), lambda bb, r: (bb, r, 0, 0)),
        out_shape=jax.ShapeDtypeStruct((B, NT * Th, Wo, N), jnp.float32),
    )(xs, w, b)


def _pack_w_e1(w):
    Ci, Co = w.shape[2], w.shape[3]
    z = jnp.zeros((Ci, Co), w.dtype)
    blocks = []
    for a in (0, 1):
        for bb in (0, 1):
            for ry in range(4):
                for rx in range(4):
                    cols = []
                    for u in (0, 1):
                        for v in (0, 1):
                            ky = ry + 4 * a - 2 * u
                            kx = rx + 4 * bb - 2 * v
                            ok = 0 <= ky <= 2 and 0 <= kx <= 2
                            cols.append(w[ky, kx] if ok else z)
                    blocks.append(jnp.concatenate(cols, axis=1))
    return jnp.concatenate(blocks, axis=0)


def _conv_e1(x, w, b, Th):
    B, H, W, C = x.shape
    HB = H // 4
    xpad = jnp.pad(x, ((0, 0), (0, 4), (0, 4), (0, 0)))
    xq = xpad.reshape(B, HB + 1, 4, HB + 1, 4, C)
    xq = xq.transpose(0, 1, 3, 2, 4, 5).reshape(B, HB + 1, HB + 1, 16 * C)
    xs = _halo_tiles(xq, Th, 1)
    y = _tap_conv(xs, _pack_w_e1(w), jnp.tile(b, 4).reshape(1, -1),
                  _TAPS2, HB, True)
    return _unphase2(y, w.shape[3])


def _pack_x_s2(x):
    B, H, W, C = x.shape
    Ho, Wo = H // 2, W // 2
    xpad = jnp.pad(x, ((0, 0), (0, 2), (0, 2), (0, 0)))
    xp = xpad.reshape(B, Ho + 1, 2, Wo + 1, 2, C)
    return xp.transpose(0, 1, 3, 2, 4, 5).reshape(B, Ho + 1, Wo + 1, 4 * C)


def _pack_w_s2(w):
    Ci, Co = w.shape[2], w.shape[3]
    z = jnp.zeros((Ci, Co), w.dtype)
    rows = []
    for a in (0, 1):
        for bb in (0, 1):
            for py in (0, 1):
                for px in (0, 1):
                    ky, kx = 2 * a + py, 2 * bb + px
                    rows.append(w[ky, kx] if (ky < 3 and kx < 3) else z)
    return jnp.concatenate(rows, axis=0)


def _conv_s2(x, w, b, Th):
    Ho, Wo = x.shape[1] // 2, x.shape[2] // 2
    xs = _halo_tiles(_pack_x_s2(x), Th, 1)
    return _tap_conv(xs, _pack_w_s2(w), b.reshape(1, -1), _TAPS2, Wo, True)


def _pack_w_t(w):
    Ci, Co = w.shape[2], w.shape[3]
    z = jnp.zeros((Ci, Co), w.dtype)
    blocks = []
    for a in (0, 1):
        for bb in (0, 1):
            cols = []
            for py in (0, 1):
                for px in (0, 1):
                    vy = (py == 0) or (a == 1)
                    vx = (px == 0) or (bb == 1)
                    ky = 2 * a if py == 0 else 1
                    kx = 2 * bb if px == 0 else 1
                    cols.append(w[ky, kx] if (vy and vx) else z)
            blocks.append(jnp.concatenate(cols, axis=1))
    return jnp.concatenate(blocks, axis=0)


def _conv_t2(x, w, b, Th):
    B, H, W, C = x.shape
    Co = w.shape[3]
    xpad = jnp.pad(x, ((0, 0), (1, 0), (1, 0), (0, 0)))
    xs = _halo_tiles(xpad, Th, 1)
    y = _tap_conv(xs, _pack_w_t(w), jnp.tile(b, 4).reshape(1, -1),
                  _TAPS2, W, True)
    return _unphase2(y, Co)


def _vq_indices(h, w4, b4, cb, Tr):
    B, Hh, Wh, C = h.shape
    D = w4.shape[1]
    K = cb.shape[1]
    NT = Hh // Tr
    M = Tr * Wh

    def body(x_ref, w_ref, b_ref, cb_ref, o_ref):
        flat = x_ref[0].reshape(M, C)
        z = lax.dot_general(flat.astype(jnp.bfloat16),
                            w_ref[...].astype(jnp.bfloat16),
                            (((1,), (0,)), ((), ())),
                            preferred_element_type=jnp.float32) \
            + b_ref[0, :][None, :]
        cbv = cb_ref[...]
        sim = lax.dot_general(z.astype(jnp.bfloat16),
                              cbv.astype(jnp.bfloat16),
                              (((1,), (0,)), ((), ())),
                              preferred_element_type=jnp.float32)
        zn = jnp.sum(z * z, axis=1, keepdims=True)
        cn = jnp.sum(cbv * cbv, axis=0)[None, :]
        dist = zn + cn - 2.0 * sim
        mn = jnp.min(dist, axis=1, keepdims=True)
        iota = lax.broadcasted_iota(jnp.int32, (M, K), 1)
        idx = jnp.min(jnp.where(dist <= mn, iota, K), axis=1)
        o_ref[0, 0, 0] = idx

    return pl.pallas_call(
        body,
        grid=(B, NT),
        in_specs=[
            pl.BlockSpec((1, Tr, Wh, C), lambda bb, r: (bb, r, 0, 0)),
            pl.BlockSpec((C, D), lambda bb, r: (0, 0)),
            pl.BlockSpec((1, D), lambda bb, r: (0, 0)),
            pl.BlockSpec((D, K), lambda bb, r: (0, 0)),
        ],
        out_specs=pl.BlockSpec((1, 1, 1, M), lambda bb, r: (bb, r, 0, 0)),
        out_shape=jax.ShapeDtypeStruct((B, NT, 1, M), jnp.int32),
    )(h, w4, b4.reshape(1, -1), cb)


def _sc_gather(table, idx):
    V, D = table.shape
    B = idx.shape[0]
    info = plsc.get_sparse_core_info()
    NC, NS = info.num_cores, info.num_subcores
    NW = NC * NS
    bpw = B // NW
    CH = 128
    nch = bpw // CH
    mesh = plsc.VectorSubcoreMesh(core_axis_name="c", subcore_axis_name="s")

    @functools.partial(
        pl.kernel, mesh=mesh,
        out_type=jax.ShapeDtypeStruct((B, D), jnp.float32),
        scratch_types=[
            pltpu.VMEM((bpw,), jnp.int32),
            pltpu.VMEM((bpw, D), jnp.float32),
            pltpu.SemaphoreType.DMA,
        ],
    )
    def k(table_hbm, idx_hbm, out_hbm, idx_v, rows_v, sem):
        wid = lax.axis_index("s") * NC + lax.axis_index("c")
        base = wid * bpw
        pltpu.sync_copy(idx_hbm.at[pl.ds(base, bpw)], idx_v)
        copies = []
        for c in range(nch):
            copies.append(pltpu.async_copy(
                table_hbm.at[idx_v.at[pl.ds(c * CH, CH)]],
                rows_v.at[pl.ds(c * CH, CH)], sem))
        for cp in copies:
            cp.wait()
        pltpu.sync_copy(rows_v, out_hbm.at[pl.ds(base, bpw)])

    return k(table, idx)


def _conv_final(x, w, b, Th):
    B, H, W, C = x.shape
    xpad = jnp.pad(x, ((0, 0), (1, 1), (1, 1), (0, 0)))
    xs = _halo_tiles(xpad, Th, 2)
    NT = xs.shape[1]
    wf = w[:, :, :, 0].reshape(9, C)

    def body(x_ref, w_ref, b_ref, o_ref):
        acc = jnp.zeros((Th, W, C), jnp.float32)
        t = 0
        for dy in range(3):
            for dx in range(3):
                xsl = x_ref[0, 0, dy:dy + Th, dx:dx + W, :]
                acc = acc + xsl * w_ref[t][None, None, :]
                t += 1
        o_ref[0] = jnp.sum(acc, axis=-1) + b_ref[0, 0]

    y = pl.pallas_call(
        body,
        grid=(B, NT),
        in_specs=[
            pl.BlockSpec((1, 1, Th + 2, W + 2, C),
                         lambda bb, r: (bb, r, 0, 0, 0)),
            pl.BlockSpec((9, C), lambda bb, r: (0, 0)),
            pl.BlockSpec((1, 1), lambda bb, r: (0, 0)),
        ],
        out_specs=pl.BlockSpec((1, Th, W), lambda bb, r: (bb, r, 0)),
        out_shape=jax.ShapeDtypeStruct((B, H, W), jnp.float32),
    )(xs, wf, b.reshape(1, 1))
    return y[..., None]


def kernel(x, We1, be1, We2, be2, We3, be3, We4, be4, codebook,
           Wd1, bd1, Wd2, bd2, Wd3, bd3, Wd4, bd4):
    h = _conv_e1(x, We1, be1, Th=32)
    h = _conv_s2(h, We2, be2, Th=32)
    h = _conv_s2(h, We3, be3, Th=32)
    idx = _vq_indices(h, We4.reshape(64, -1), be4, codebook, Tr=16)
    B = x.shape[0]
    D = codebook.shape[0]
    idx_flat = idx.reshape(-1)
    table = jnp.pad(codebook.T, ((0, 0), (0, 128 - D)))
    q = _sc_gather(table, idx_flat)
    q = q[:, :D].reshape(B, 64, 64, D)
    h = _conv_t2(q, Wd1, bd1, Th=32)
    h = _conv_t2(h, Wd2, bd2, Th=32)
    h = _conv_t2(h, Wd3, bd3, Th=16)
    out = _conv_final(h, Wd4, bd4, Th=32)
    return out
